# baseline (speedup 1.0000x reference)
import functools

import jax
import jax.numpy as jnp
from jax import lax
from jax.experimental import pallas as pl
from jax.experimental.pallas import tpu as pltpu
from jax.experimental.pallas import tpu_sc as plsc

B, T, H = 4096, 200, 64
N = B * T
NC, NS = 2, 16
NW = NC * NS
PER_W = N // NW
CHUNK = 128
NCHUNK = PER_W // CHUNK

_mesh = plsc.VectorSubcoreMesh(core_axis_name="c", subcore_axis_name="s")


@functools.partial(
    pl.kernel,
    mesh=_mesh,
    out_type=jax.ShapeDtypeStruct((N, H), jnp.float32),
    compiler_params=pltpu.CompilerParams(use_tc_tiling_on_sc=False),
    scratch_types=[
        pltpu.VMEM((NCHUNK, CHUNK), jnp.int32),
        pltpu.VMEM((CHUNK, H), jnp.float32),
        pltpu.VMEM((CHUNK, H), jnp.float32),
        pltpu.SemaphoreType.DMA,
        pltpu.SemaphoreType.DMA,
        pltpu.SemaphoreType.DMA,
        pltpu.SemaphoreType.DMA,
    ],
)
def _embed_sc(ids_hbm, w_hbm, out_hbm, idx_v, rows0, rows1, g0sem, g1sem,
              s0sem, s1sem):
    wid = lax.axis_index("s") * NC + lax.axis_index("c")
    base = wid * PER_W
    pltpu.sync_copy(ids_hbm.at[wid], idx_v)

    def step2(j, carry):
        i0 = j * 2
        i1 = i0 + 1
        g0 = pltpu.async_copy(w_hbm.at[idx_v.at[i0]], rows0, g0sem)
        g1 = pltpu.async_copy(w_hbm.at[idx_v.at[i1]], rows1, g1sem)
        g0.wait()
        s0 = pltpu.async_copy(
            rows0, out_hbm.at[pl.ds(base + i0 * CHUNK, CHUNK)], s0sem)
        g1.wait()
        s1 = pltpu.async_copy(
            rows1, out_hbm.at[pl.ds(base + i1 * CHUNK, CHUNK)], s1sem)
        s0.wait()
        s1.wait()
        return carry

    lax.fori_loop(0, NCHUNK // 2, step2, 0)


def kernel(input_ids, weight):
    ids = input_ids.astype(jnp.int32).reshape(NW, NCHUNK, CHUNK)
    out = _embed_sc(ids, weight)
    return out.reshape(B, T, H)


# speedup vs baseline: 1.0358x; 1.0358x over previous
import functools

import jax
import jax.numpy as jnp
from jax import lax
from jax.experimental import pallas as pl
from jax.experimental.pallas import tpu as pltpu
from jax.experimental.pallas import tpu_sc as plsc

B, T, H = 4096, 200, 64
N = B * T
NC, NS = 2, 16
NW = NC * NS
PER_W = N // NW
CHUNK = 128
NCHUNK = PER_W // CHUNK
KPB = 4
ROWS = KPB * CHUNK
NITER = NCHUNK // (2 * KPB)

_mesh = plsc.VectorSubcoreMesh(core_axis_name="c", subcore_axis_name="s")


@functools.partial(
    pl.kernel,
    mesh=_mesh,
    out_type=jax.ShapeDtypeStruct((N, H), jnp.float32),
    compiler_params=pltpu.CompilerParams(use_tc_tiling_on_sc=False),
    scratch_types=[
        pltpu.VMEM((NCHUNK, CHUNK), jnp.int32),
        pltpu.VMEM((ROWS, H), jnp.float32),
        pltpu.VMEM((ROWS, H), jnp.float32),
        pltpu.SemaphoreType.DMA,
        pltpu.SemaphoreType.DMA,
        pltpu.SemaphoreType.DMA,
        pltpu.SemaphoreType.DMA,
    ],
)
def _embed_sc(ids_hbm, w_hbm, out_hbm, idx_v, rows_a, rows_b, gsa, gsb,
              ssa, ssb):
    wid = lax.axis_index("s") * NC + lax.axis_index("c")
    base = wid * PER_W

    pltpu.sync_copy(ids_hbm.at[wid], idx_v)

    def gathers(buf, gsem, chunk0):
        cps = []
        for k in range(KPB):
            cps.append(pltpu.async_copy(
                w_hbm.at[idx_v.at[chunk0 + k]],
                buf.at[pl.ds(k * CHUNK, CHUNK)], gsem))
        return cps

    def step(j, carry):
        c0 = j * 2 * KPB
        off_a = base + c0 * CHUNK
        off_b = off_a + ROWS
        prev_a = off_a - 2 * ROWS
        prev_b = off_b - 2 * ROWS

        @pl.when(j > 0)
        def _():
            pltpu.make_async_copy(
                rows_a, out_hbm.at[pl.ds(prev_a, ROWS)], ssa).wait()

        ga = gathers(rows_a, gsa, c0)

        @pl.when(j > 0)
        def _():
            pltpu.make_async_copy(
                rows_b, out_hbm.at[pl.ds(prev_b, ROWS)], ssb).wait()

        gb = gathers(rows_b, gsb, c0 + KPB)

        for cp in ga:
            cp.wait()
        pltpu.async_copy(rows_a, out_hbm.at[pl.ds(off_a, ROWS)], ssa)
        for cp in gb:
            cp.wait()
        pltpu.async_copy(rows_b, out_hbm.at[pl.ds(off_b, ROWS)], ssb)
        return carry

    lax.fori_loop(0, NITER, step, 0)

    last_a = base + (NITER - 1) * 2 * ROWS
    pltpu.make_async_copy(
        rows_a, out_hbm.at[pl.ds(last_a, ROWS)], ssa).wait()
    pltpu.make_async_copy(
        rows_b, out_hbm.at[pl.ds(last_a + ROWS, ROWS)], ssb).wait()


def kernel(input_ids, weight):
    ids = input_ids.astype(jnp.int32).reshape(NW, NCHUNK, CHUNK)
    out = _embed_sc(ids, weight)
    return out.reshape(B, T, H)
